# baseline (speedup 1.0000x reference)
import functools

import jax
import jax.numpy as jnp
from jax import lax
from jax.experimental import pallas as pl
from jax.experimental.pallas import tpu as pltpu
from jax.experimental.pallas import tpu_sc as plsc

NC = 2
NS = 16
NW = NC * NS

CH = 128
GRP = 8


def _sc_aggregate(x, srcs, dsts, n_chunks, n_pad, d, rows_per_tile):
    mesh = plsc.VectorSubcoreMesh(core_axis_name="c", subcore_axis_name="s")
    n_seg2 = n_chunks // (2 * GRP)

    @functools.partial(
        pl.kernel,
        mesh=mesh,
        out_type=(
            jax.ShapeDtypeStruct((NC, n_pad, d), jnp.float32),
            jax.ShapeDtypeStruct((NC * n_pad,), jnp.float32),
        ),
        scratch_types=[
            pltpu.VMEM((GRP, CH), jnp.int32),
            pltpu.VMEM((GRP, CH), jnp.int32),
            pltpu.VMEM((GRP, CH), jnp.int32),
            pltpu.VMEM((GRP, CH), jnp.int32),
            pltpu.VMEM((CH, 128), jnp.float32),
            pltpu.VMEM((CH, 128), jnp.float32),
            pltpu.VMEM((CH,), jnp.float32),
            pltpu.VMEM_SHARED((n_pad, 128), jnp.float32),
            pltpu.VMEM_SHARED((n_pad,), jnp.float32),
            pltpu.SemaphoreType.DMA,
            pltpu.SemaphoreType.DMA,
            pltpu.SemaphoreType.DMA,
            pltpu.SemaphoreType.DMA,
            pltpu.SemaphoreType.DMA,
            pltpu.SemaphoreType.DMA,
            pltpu.SemaphoreType.DMA,
            pltpu.SemaphoreType.DMA,
        ],
    )
    def body(x_hbm, srcs_hbm, dsts_hbm, parts_hbm, cnts_hbm,
             src0, dst0, src1, dst1, buf_a, buf_b, ones_v, acc, cnt_acc,
             sem_ga, sem_gb, sem_sa, sem_sb, sem_i0, sem_i1,
             sem_c0, sem_c1):
        cid = lax.axis_index("c")
        sid = lax.axis_index("s")
        wid = sid * NC + cid

        src_b = (src0, src1)
        dst_b = (dst0, dst1)
        sem_i = (sem_i0, sem_i1)
        bufs = (buf_a, buf_b)
        gsem = (sem_ga, sem_gb)
        ssem = (sem_sa, sem_sb)
        csem = (sem_c0, sem_c1)

        z16 = jnp.zeros((16,), jnp.float32)
        o16 = jnp.ones((16,), jnp.float32)
        for i in range(CH // 16):
            ones_v[pl.ds(i * 16, 16)] = o16

        def zrow(r, carry):
            for i in range(128 // 16):
                buf_a[r, pl.ds(i * 16, 16)] = z16
            return carry

        lax.fori_loop(0, CH, zrow, 0)

        row0 = sid * rows_per_tile
        for r in range(rows_per_tile // CH):
            pltpu.sync_copy(buf_a, acc.at[pl.ds(row0 + r * CH, CH)])
            pltpu.sync_copy(buf_a.at[0],
                            cnt_acc.at[pl.ds(row0 + r * CH, CH)])
        plsc.subcore_barrier()

        def stage_start(seg, bank):
            pltpu.make_async_copy(
                srcs_hbm.at[wid, pl.ds(seg * GRP, GRP)], src_b[bank],
                sem_i[bank]).start()
            pltpu.make_async_copy(
                dsts_hbm.at[wid, pl.ds(seg * GRP, GRP)], dst_b[bank],
                sem_i[bank]).start()

        def stage_wait(seg, bank):
            pltpu.make_async_copy(
                srcs_hbm.at[wid, pl.ds(seg * GRP, GRP)], src_b[bank],
                sem_i[bank]).wait()
            pltpu.make_async_copy(
                dsts_hbm.at[wid, pl.ds(seg * GRP, GRP)], dst_b[bank],
                sem_i[bank]).wait()

        def g_start(bank, row, db):
            pltpu.make_async_copy(
                x_hbm.at[src_b[bank].at[row]], bufs[db], gsem[db]).start()

        def g_wait(bank, row, db):
            pltpu.make_async_copy(
                x_hbm.at[src_b[bank].at[row]], bufs[db], gsem[db]).wait()

        def s_start(bank, row, db):
            pltpu.async_copy(
                bufs[db], acc.at[dst_b[bank].at[row]], ssem[db], add=True)

        def s_wait(bank, row, db):
            pltpu.make_async_copy(
                bufs[db], acc.at[dst_b[bank].at[row]], ssem[db]).wait()

        def cnt_start(bank, row):
            pltpu.async_copy(
                ones_v, cnt_acc.at[dst_b[bank].at[row]], csem[bank],
                add=True)

        def cnt_wait(bank, row):
            pltpu.make_async_copy(
                ones_v, cnt_acc.at[dst_b[bank].at[row]], csem[bank]).wait()

        stage_start(0, 0)
        stage_wait(0, 0)
        g_start(0, 0, 0)

        def seg_pair(g2, carry):
            for k in range(2 * GRP):
                bank = k // GRP
                b = k % GRP
                db = k % 2
                pb, pbank = (b - 1, bank) if b else (GRP - 1, 1 - bank)

                g_wait(bank, b, db)
                s_start(bank, b, db)

                if k == 0:
                    @pl.when(g2 > 0)
                    def _():
                        s_wait(pbank, pb, 1 - db)
                else:
                    s_wait(pbank, pb, 1 - db)

                if k == 2:
                    @pl.when(g2 > 0)
                    def _():
                        for r in range(GRP):
                            cnt_wait(1, r)

                    stage_start(2 * g2 + 1, 1)
                elif k == GRP + 2:
                    for r in range(GRP):
                        cnt_wait(0, r)

                    @pl.when(g2 < n_seg2 - 1)
                    def _():
                        stage_start(2 * g2 + 2, 0)

                cnt_start(bank, b)

                if k + 1 < GRP:
                    g_start(bank, b + 1, 1 - db)
                elif k + 1 == GRP:
                    stage_wait(2 * g2 + 1, 1)
                    g_start(1, 0, 1 - db)
                elif k + 1 < 2 * GRP:
                    g_start(bank, b + 1, 1 - db)
                else:
                    @pl.when(g2 < n_seg2 - 1)
                    def _():
                        stage_wait(2 * g2 + 2, 0)
                        g_start(0, 0, 1 - db)

            return carry

        lax.fori_loop(0, n_seg2, seg_pair, 0)

        s_wait(1, GRP - 1, 1)
        for r in range(GRP):
            cnt_wait(1, r)

        plsc.subcore_barrier()
        pltpu.sync_copy(acc.at[pl.ds(row0, rows_per_tile)],
                        parts_hbm.at[cid, pl.ds(row0, rows_per_tile)])
        pltpu.sync_copy(cnt_acc.at[pl.ds(row0, rows_per_tile)],
                        cnts_hbm.at[pl.ds(cid * n_pad + row0, rows_per_tile)])

    return body(x, srcs, dsts)


def _tc_right(x, W_r):
    def body(x_ref, wr_ref, o_ref):
        o_ref[...] = jnp.dot(x_ref[...], wr_ref[...],
                             preferred_element_type=jnp.float32,
                             precision=lax.Precision.HIGHEST)

    return pl.pallas_call(
        body,
        out_shape=jax.ShapeDtypeStruct(x.shape, jnp.float32),
    )(x, W_r)


def _tc_combine(parts, cnts, xr, W_l, b2, n, d):
    def body(p_ref, c_ref, xr_ref, wl_ref, b_ref, o_ref):
        p = p_ref[0, :n] + p_ref[1, :n]
        n_pad = c_ref.shape[0] // 2
        c = c_ref[:n_pad] + c_ref[n_pad:]
        c2 = jnp.maximum(c[:n], 1.0).reshape(n, 1)
        mean = p / c2
        acc = jnp.dot(mean, wl_ref[...], preferred_element_type=jnp.float32,
                      precision=lax.Precision.HIGHEST)
        o_ref[...] = jnp.maximum(acc + xr_ref[...] + b_ref[...], 0.0)

    return pl.pallas_call(
        body,
        out_shape=jax.ShapeDtypeStruct((n, d), jnp.float32),
    )(parts, cnts, xr, W_l, b2)


def kernel(x, edge_index, W_l, b_l, W_r):
    n, d = x.shape
    e = edge_index.shape[1]
    rows_per_tile = -(-(n + 1) // (NS * CH)) * CH
    n_pad = rows_per_tile * NS

    n_chunks = -(-e // (NW * CH))
    n_chunks = -(-n_chunks // (2 * GRP)) * (2 * GRP)
    e_pad = NW * CH * n_chunks
    npd = e_pad - e
    pad_src = (jnp.arange(npd, dtype=jnp.int32) * 37) % n
    pad_dst = n + (jnp.arange(npd, dtype=jnp.int32) % (n_pad - n))
    src = jnp.concatenate([edge_index[0], pad_src]).reshape(NW, n_chunks, CH)
    dst = jnp.concatenate([edge_index[1], pad_dst]).reshape(NW, n_chunks, CH)

    parts, cnts = _sc_aggregate(x, src, dst, n_chunks, n_pad, d,
                                rows_per_tile)
    xr = _tc_right(x, W_r)

    b2 = b_l.reshape(1, d)
    return _tc_combine(parts, cnts, xr, W_l, b2, n, d)


# speedup vs baseline: 1.0315x; 1.0315x over previous
import functools

import jax
import jax.numpy as jnp
from jax import lax
from jax.experimental import pallas as pl
from jax.experimental.pallas import tpu as pltpu
from jax.experimental.pallas import tpu_sc as plsc

NC = 2
NS = 16
NW = NC * NS

CH = 64
GRP = 8


def _sc_aggregate(x, srcs, dsts, n_chunks, n_pad, d, rows_per_tile):
    mesh = plsc.VectorSubcoreMesh(core_axis_name="c", subcore_axis_name="s")
    n_seg2 = n_chunks // (2 * GRP)

    @functools.partial(
        pl.kernel,
        mesh=mesh,
        out_type=(
            jax.ShapeDtypeStruct((NC, n_pad, d), jnp.float32),
            jax.ShapeDtypeStruct((NC * n_pad,), jnp.float32),
        ),
        scratch_types=[
            pltpu.VMEM((GRP, CH), jnp.int32),
            pltpu.VMEM((GRP, CH), jnp.int32),
            pltpu.VMEM((GRP, CH), jnp.int32),
            pltpu.VMEM((GRP, CH), jnp.int32),
            pltpu.VMEM((CH, 128), jnp.float32),
            pltpu.VMEM((CH, 128), jnp.float32),
            pltpu.VMEM((CH, 128), jnp.float32),
            pltpu.VMEM((CH, 128), jnp.float32),
            pltpu.VMEM((CH,), jnp.float32),
            pltpu.VMEM_SHARED((n_pad, 128), jnp.float32),
            pltpu.VMEM_SHARED((n_pad,), jnp.float32),
            pltpu.SemaphoreType.DMA,
            pltpu.SemaphoreType.DMA,
            pltpu.SemaphoreType.DMA,
            pltpu.SemaphoreType.DMA,
            pltpu.SemaphoreType.DMA,
            pltpu.SemaphoreType.DMA,
            pltpu.SemaphoreType.DMA,
            pltpu.SemaphoreType.DMA,
            pltpu.SemaphoreType.DMA,
            pltpu.SemaphoreType.DMA,
            pltpu.SemaphoreType.DMA,
            pltpu.SemaphoreType.DMA,
        ],
    )
    def body(x_hbm, srcs_hbm, dsts_hbm, parts_hbm, cnts_hbm,
             src0, dst0, src1, dst1, buf_a, buf_b, buf_c, buf_d,
             ones_v, acc, cnt_acc,
             sem_ga, sem_gb, sem_gc, sem_gd,
             sem_sa, sem_sb, sem_sc, sem_sd, sem_i0, sem_i1,
             sem_c0, sem_c1):
        cid = lax.axis_index("c")
        sid = lax.axis_index("s")
        wid = sid * NC + cid

        src_b = (src0, src1)
        dst_b = (dst0, dst1)
        sem_i = (sem_i0, sem_i1)
        bufs = (buf_a, buf_b, buf_c, buf_d)
        gsem = (sem_ga, sem_gb, sem_gc, sem_gd)
        ssem = (sem_sa, sem_sb, sem_sc, sem_sd)
        csem = (sem_c0, sem_c1)

        z16 = jnp.zeros((16,), jnp.float32)
        o16 = jnp.ones((16,), jnp.float32)
        for i in range(CH // 16):
            ones_v[pl.ds(i * 16, 16)] = o16

        def zrow(r, carry):
            for i in range(128 // 16):
                buf_a[r, pl.ds(i * 16, 16)] = z16
            return carry

        lax.fori_loop(0, CH, zrow, 0)

        row0 = sid * rows_per_tile
        for r in range(rows_per_tile // CH):
            pltpu.sync_copy(buf_a, acc.at[pl.ds(row0 + r * CH, CH)])
        for r in range(rows_per_tile // 128):
            pltpu.sync_copy(buf_a.at[0],
                            cnt_acc.at[pl.ds(row0 + r * 128, 128)])
        plsc.subcore_barrier()

        def stage_start(seg, bank):
            pltpu.make_async_copy(
                srcs_hbm.at[wid, pl.ds(seg * GRP, GRP)], src_b[bank],
                sem_i[bank]).start()
            pltpu.make_async_copy(
                dsts_hbm.at[wid, pl.ds(seg * GRP, GRP)], dst_b[bank],
                sem_i[bank]).start()

        def stage_wait(seg, bank):
            pltpu.make_async_copy(
                srcs_hbm.at[wid, pl.ds(seg * GRP, GRP)], src_b[bank],
                sem_i[bank]).wait()
            pltpu.make_async_copy(
                dsts_hbm.at[wid, pl.ds(seg * GRP, GRP)], dst_b[bank],
                sem_i[bank]).wait()

        def g_start(bank, row, db):
            pltpu.make_async_copy(
                x_hbm.at[src_b[bank].at[row]], bufs[db], gsem[db]).start()

        def g_wait(bank, row, db):
            pltpu.make_async_copy(
                x_hbm.at[src_b[bank].at[row]], bufs[db], gsem[db]).wait()

        def s_start(bank, row, db):
            pltpu.async_copy(
                bufs[db], acc.at[dst_b[bank].at[row]], ssem[db], add=True)

        def s_wait(bank, row, db):
            pltpu.make_async_copy(
                bufs[db], acc.at[dst_b[bank].at[row]], ssem[db]).wait()

        def cnt_start(bank, row):
            pltpu.async_copy(
                ones_v, cnt_acc.at[dst_b[bank].at[row]], csem[bank],
                add=True)

        def cnt_wait(bank, row):
            pltpu.make_async_copy(
                ones_v, cnt_acc.at[dst_b[bank].at[row]], csem[bank]).wait()

        stage_start(0, 0)
        stage_wait(0, 0)
        g_start(0, 0, 0)
        g_start(0, 1, 1)

        def seg_pair(g2, carry):
            for k in range(2 * GRP):
                bank = k // GRP
                b = k % GRP
                db = k % 4

                g_wait(bank, b, db)
                s_start(bank, b, db)

                if k <= 1:
                    @pl.when(g2 > 0)
                    def _():
                        s_wait(1, GRP - 2 + k, (k + 2) % 4)
                else:
                    j = k - 2
                    s_wait(j // GRP, j % GRP, j % 4)

                if k == 2:
                    @pl.when(g2 > 0)
                    def _():
                        for r in range(GRP):
                            cnt_wait(1, r)

                    stage_start(2 * g2 + 1, 1)
                elif k == GRP + 2:
                    for r in range(GRP):
                        cnt_wait(0, r)

                    @pl.when(g2 < n_seg2 - 1)
                    def _():
                        stage_start(2 * g2 + 2, 0)

                cnt_start(bank, b)

                jn = k + 2
                if jn < GRP:
                    g_start(0, jn, jn % 4)
                elif jn == GRP:
                    stage_wait(2 * g2 + 1, 1)
                    g_start(1, 0, jn % 4)
                elif jn < 2 * GRP:
                    g_start(1, jn - GRP, jn % 4)
                elif jn == 2 * GRP:
                    @pl.when(g2 < n_seg2 - 1)
                    def _():
                        stage_wait(2 * g2 + 2, 0)
                        g_start(0, 0, jn % 4)
                else:
                    @pl.when(g2 < n_seg2 - 1)
                    def _():
                        g_start(0, 1, jn % 4)

            return carry

        lax.fori_loop(0, n_seg2, seg_pair, 0)

        s_wait(1, GRP - 2, (2 * GRP - 2) % 4)
        s_wait(1, GRP - 1, (2 * GRP - 1) % 4)
        for r in range(GRP):
            cnt_wait(1, r)

        plsc.subcore_barrier()
        pltpu.sync_copy(acc.at[pl.ds(row0, rows_per_tile)],
                        parts_hbm.at[cid, pl.ds(row0, rows_per_tile)])
        pltpu.sync_copy(cnt_acc.at[pl.ds(row0, rows_per_tile)],
                        cnts_hbm.at[pl.ds(cid * n_pad + row0, rows_per_tile)])

    return body(x, srcs, dsts)


def _tc_right(x, W_r):
    def body(x_ref, wr_ref, o_ref):
        o_ref[...] = jnp.dot(x_ref[...], wr_ref[...],
                             preferred_element_type=jnp.float32,
                             precision=lax.Precision.HIGHEST)

    return pl.pallas_call(
        body,
        out_shape=jax.ShapeDtypeStruct(x.shape, jnp.float32),
    )(x, W_r)


def _tc_combine(parts, cnts, xr, W_l, b2, n, d):
    def body(p_ref, c_ref, xr_ref, wl_ref, b_ref, o_ref):
        p = p_ref[0, :n] + p_ref[1, :n]
        n_pad = c_ref.shape[0] // 2
        c = c_ref[:n_pad] + c_ref[n_pad:]
        c2 = jnp.maximum(c[:n], 1.0).reshape(n, 1)
        mean = p / c2
        acc = jnp.dot(mean, wl_ref[...], preferred_element_type=jnp.float32,
                      precision=lax.Precision.HIGHEST)
        o_ref[...] = jnp.maximum(acc + xr_ref[...] + b_ref[...], 0.0)

    return pl.pallas_call(
        body,
        out_shape=jax.ShapeDtypeStruct((n, d), jnp.float32),
    )(parts, cnts, xr, W_l, b2)


def kernel(x, edge_index, W_l, b_l, W_r):
    n, d = x.shape
    e = edge_index.shape[1]
    rows_per_tile = -(-(n + 1) // (NS * CH)) * CH
    n_pad = rows_per_tile * NS

    n_chunks = -(-e // (NW * CH))
    n_chunks = -(-n_chunks // (2 * GRP)) * (2 * GRP)
    e_pad = NW * CH * n_chunks
    npd = e_pad - e
    pad_src = (jnp.arange(npd, dtype=jnp.int32) * 37) % n
    pad_dst = n + (jnp.arange(npd, dtype=jnp.int32) % (n_pad - n))
    src = jnp.concatenate([edge_index[0], pad_src]).reshape(NW, n_chunks, CH)
    dst = jnp.concatenate([edge_index[1], pad_dst]).reshape(NW, n_chunks, CH)

    parts, cnts = _sc_aggregate(x, src, dst, n_chunks, n_pad, d,
                                rows_per_tile)
    xr = _tc_right(x, W_r)

    b2 = b_l.reshape(1, d)
    return _tc_combine(parts, cnts, xr, W_l, b2, n, d)
